# baseline (speedup 1.0000x reference)
import functools

import jax
import jax.numpy as jnp
from jax import lax
from jax.experimental import pallas as pl
from jax.experimental.pallas import tpu as pltpu
from jax.experimental.pallas import tpu_sc as plsc

VOXEL_SIZE = (0.1, 0.1, 0.2)
PC_RANGE = (-54.0, -54.0, -5.0)
DOWNSAMPLE = 16.0
DEPTH_MIN, DEPTH_MAX = 1.0, 60.0

LANES = 128
V_CHUNK = 128
NZROWS = 128


def _idx_body(ncam, fh, fw, zrow, pi_ref, idx_ref):
    shape = pi_ref.shape[1:]
    spread = zrow + (
        lax.broadcasted_iota(jnp.int32, shape, 0) * LANES
        + lax.broadcasted_iota(jnp.int32, shape, 1)) % NZROWS
    for c in range(ncam):
        rx = pi_ref[3 * c]
        ry = pi_ref[3 * c + 1]
        rz = pi_ref[3 * c + 2]
        cu = jnp.round(rx / DOWNSAMPLE)
        cv = jnp.round(ry / DOWNSAMPLE)
        kept = ((cu >= 0.0) & (cu < float(fw)) & (cv >= 0.0) & (cv < float(fh))
                & (rz < DEPTH_MAX) & (rz >= DEPTH_MIN))
        ci = jnp.clip(cu.astype(jnp.int32), 0, fw - 1)
        cj = jnp.clip(cv.astype(jnp.int32), 0, fh - 1)
        flat = cj * fw + ci + c * (fh * fw)
        idx_ref[c] = jnp.where(kept, flat, spread)


def _table_body(img_ref, w_ref, out_ref):
    a = img_ref[0]
    w = w_ref[...]
    t = lax.dot_general(a, w, (((0,), (1,)), ((), ())),
                        preferred_element_type=jnp.float32)
    out_ref[0] = t.astype(jnp.bfloat16)


def _fuse_body(vf_ref, im_ref, w1_ref, w2_ref, out_ref):
    out_ref[...] = (
        jnp.dot(vf_ref[...], w1_ref[...], preferred_element_type=jnp.float32)
        + jnp.dot(im_ref[...], w2_ref[...], preferred_element_type=jnp.float32))


def _sc_gather_body(ncam, nc, ns, k_chunks,
                    idx_hbm, table_hbm, out_hbm, idxv, rows, sems, osems):
    wid = lax.axis_index("s") * nc + lax.axis_index("c")
    kv = k_chunks * V_CHUNK
    base = wid * kv
    pltpu.sync_copy(idx_hbm.at[:, pl.ds(base, kv)], idxv)

    def fire(g):
        buf = g % 2
        return [pltpu.async_copy(
                    table_hbm.at[idxv.at[c, pl.ds(g * V_CHUNK, V_CHUNK)]],
                    rows[buf][c], sems[buf][c])
                for c in range(ncam)]

    pending_out = [None, None]
    cur = fire(0)
    for g in range(k_chunks):
        buf = g % 2
        for cp in cur:
            cp.wait()
        nxt = None
        if g + 1 < k_chunks:
            nbuf = (g + 1) % 2
            if pending_out[nbuf] is not None:
                pending_out[nbuf].wait()
                pending_out[nbuf] = None
            nxt = fire(g + 1)

        def accum(j, carry, _buf=buf):
            for s2 in range(2):
                sl = pl.ds(s2 * 32, 32)
                acc = rows[_buf][0][j, sl]
                for c in range(1, ncam):
                    acc = acc + rows[_buf][c][j, sl]
                rows[_buf][0][j, sl] = acc
            return carry

        lax.fori_loop(0, V_CHUNK, accum, 0, unroll=2)
        pending_out[buf] = pltpu.async_copy(
            rows[buf][0], out_hbm.at[pl.ds(base + g * V_CHUNK, V_CHUNK)],
            osems[buf])
        cur = nxt
    for po in pending_out:
        if po is not None:
            po.wait()


def kernel(voxel_features, voxel_coords, img_features, rots, trans, intrins,
           post_rots, post_trans, bda, lidar2cam, imgs, W_compress, W_fuse):
    n = voxel_features.shape[0]
    ncam = img_features.shape[1]
    fh, fw = img_features.shape[3], img_features.shape[4]
    p = fh * fw
    out_ch = W_fuse.shape[0]
    in_ch = voxel_features.shape[1]
    cmp_ch = W_compress.shape[0]
    zrow = ncam * p

    try:
        info = plsc.get_sparse_core_info()
        nc, ns = info.num_cores, info.num_subcores
    except Exception:
        nc, ns = 2, 16
    nw = nc * ns
    chunk_rows = nw * V_CHUNK
    k_chunks = -(-n // chunk_rows)
    n_pad = k_chunks * chunk_rows
    nb = n_pad // LANES

    b = 0
    pts = voxel_coords[:, jnp.array([3, 2, 1])].astype(jnp.float32)
    pts = pts * jnp.asarray(VOXEL_SIZE, jnp.float32)[None, :] \
        + jnp.asarray(PC_RANGE, jnp.float32)[None, :]
    bda_b = bda[b]
    pc = pts - bda_b[:3, 3][None, :]
    pc = pc @ jnp.linalg.inv(bda_b[:3, :3]).T
    pc_t = pc.T
    pi_rows = []
    for c in range(ncam):
        l2c = lidar2cam[b, c]
        cam2img = jnp.eye(4, dtype=jnp.float32).at[:3, :3].set(intrins[b, c])
        lidar2img = cam2img @ l2c.T
        q = lidar2img[:3, :3] @ pc_t + lidar2img[:3, 3][:, None]
        q = jnp.concatenate([q[:2] / q[2:3], q[2:3]], axis=0)
        q = post_rots[b, c] @ q + post_trans[b, c][:, None]
        pi_rows.append(q)
    pi_t = jnp.concatenate(pi_rows, axis=0)
    pi_t = jnp.pad(pi_t, ((0, 0), (0, n_pad - n)))
    pi3 = pi_t.reshape(3 * ncam, nb, LANES)

    br = 32
    grid_a = nb // br
    assert grid_a * br == nb
    idx3 = pl.pallas_call(
        functools.partial(_idx_body, ncam, fh, fw, zrow),
        grid=(grid_a,),
        in_specs=[
            pl.BlockSpec((3 * ncam, br, LANES), lambda i: (0, i, 0)),
        ],
        out_specs=pl.BlockSpec((ncam, br, LANES), lambda i: (0, i, 0)),
        out_shape=jax.ShapeDtypeStruct((ncam, nb, LANES), jnp.int32),
    )(pi3)
    idx = idx3.reshape(ncam, n_pad)

    img_flat = img_features[b].reshape(ncam, img_features.shape[2], p)
    tbl = pl.pallas_call(
        _table_body,
        grid=(ncam,),
        in_specs=[
            pl.BlockSpec((1, img_flat.shape[1], p), lambda i: (i, 0, 0)),
            pl.BlockSpec((cmp_ch, img_flat.shape[1]), lambda i: (0, 0)),
        ],
        out_specs=pl.BlockSpec((1, p, cmp_ch), lambda i: (i, 0, 0)),
        out_shape=jax.ShapeDtypeStruct((ncam, p, cmp_ch), jnp.bfloat16),
    )(img_flat, W_compress)
    table = jnp.concatenate(
        [tbl.reshape(ncam * p, cmp_ch),
         jnp.zeros((NZROWS, cmp_ch), jnp.bfloat16)], axis=0)

    mesh = plsc.VectorSubcoreMesh(core_axis_name="c", subcore_axis_name="s",
                                  num_cores=nc, num_subcores=ns)
    img_feat = pl.kernel(
        functools.partial(_sc_gather_body, ncam, nc, ns, k_chunks),
        out_type=jax.ShapeDtypeStruct((n_pad, cmp_ch), jnp.bfloat16),
        mesh=mesh,
        scratch_types=[
            pltpu.VMEM((ncam, k_chunks * V_CHUNK), jnp.int32),
            [[pltpu.VMEM((V_CHUNK, cmp_ch), jnp.bfloat16) for _ in range(ncam)]
             for _ in range(2)],
            [[pltpu.SemaphoreType.DMA for _ in range(ncam)] for _ in range(2)],
            [pltpu.SemaphoreType.DMA for _ in range(2)],
        ],
        compiler_params=pltpu.CompilerParams(use_tc_tiling_on_sc=False),
    )(idx, table)

    w1t = W_fuse[:, :in_ch].T
    w2t = W_fuse[:, in_ch:].T.astype(jnp.bfloat16)
    bn = 512
    grid_c = -(-n // bn)
    fused = pl.pallas_call(
        _fuse_body,
        grid=(grid_c,),
        in_specs=[
            pl.BlockSpec((bn, in_ch), lambda i: (i, 0)),
            pl.BlockSpec((bn, cmp_ch), lambda i: (i, 0)),
            pl.BlockSpec((in_ch, out_ch), lambda i: (0, 0)),
            pl.BlockSpec((cmp_ch, out_ch), lambda i: (0, 0)),
        ],
        out_specs=pl.BlockSpec((bn, out_ch), lambda i: (i, 0)),
        out_shape=jax.ShapeDtypeStruct((n, out_ch), jnp.float32),
    )(voxel_features, img_feat, w1t, w2t)

    return (fused, voxel_coords)


# speedup vs baseline: 1.4471x; 1.4471x over previous
import functools

import jax
import jax.numpy as jnp
from jax import lax
from jax.experimental import pallas as pl
from jax.experimental.pallas import tpu as pltpu
from jax.experimental.pallas import tpu_sc as plsc

VOXEL_SIZE = (0.1, 0.1, 0.2)
PC_RANGE = (-54.0, -54.0, -5.0)
DOWNSAMPLE = 16.0
DEPTH_MIN, DEPTH_MAX = 1.0, 60.0

LANES = 128
V_CHUNK = 128
NZROWS = 128


def _idx_body(ncam, fh, fw, zrow, pi_ref, idx_ref):
    shape = pi_ref.shape[1:]
    spread = zrow + (
        lax.broadcasted_iota(jnp.int32, shape, 0) * LANES
        + lax.broadcasted_iota(jnp.int32, shape, 1)) % NZROWS
    for c in range(ncam):
        rx = pi_ref[3 * c]
        ry = pi_ref[3 * c + 1]
        rz = pi_ref[3 * c + 2]
        cu = jnp.round(rx / DOWNSAMPLE)
        cv = jnp.round(ry / DOWNSAMPLE)
        kept = ((cu >= 0.0) & (cu < float(fw)) & (cv >= 0.0) & (cv < float(fh))
                & (rz < DEPTH_MAX) & (rz >= DEPTH_MIN))
        ci = jnp.clip(cu.astype(jnp.int32), 0, fw - 1)
        cj = jnp.clip(cv.astype(jnp.int32), 0, fh - 1)
        flat = cj * fw + ci + c * (fh * fw)
        idx_ref[c] = jnp.where(kept, flat, spread)


def _table_body(img_ref, w_ref, out_ref):
    a = img_ref[0]
    w = w_ref[...]
    t = lax.dot_general(a, w, (((0,), (1,)), ((), ())),
                        preferred_element_type=jnp.float32)
    out_ref[0] = t.astype(jnp.bfloat16)


def _fuse_body(vf_ref, im_ref, w1_ref, w2_ref, out_ref):
    out_ref[...] = (
        jnp.dot(vf_ref[...], w1_ref[...], preferred_element_type=jnp.float32)
        + jnp.dot(im_ref[...], w2_ref[...], preferred_element_type=jnp.float32))


def _sc_gather_body(ncam, nc, ns, k_chunks,
                    idx_hbm, table_hbm, out_hbm, idxv, rows, sems, osems,
                    table_sp):
    wid = lax.axis_index("s") * nc + lax.axis_index("c")
    kv = k_chunks * V_CHUNK
    base = wid * kv
    @pl.when(lax.axis_index("s") == 0)
    def _stage():
        pltpu.sync_copy(table_hbm, table_sp)

    pltpu.sync_copy(idx_hbm.at[:, pl.ds(base, kv)], idxv)
    plsc.subcore_barrier()

    def fire(g):
        buf = g % 2
        return [pltpu.async_copy(
                    table_sp.at[idxv.at[c, pl.ds(g * V_CHUNK, V_CHUNK)]],
                    rows[buf][c], sems[buf][c])
                for c in range(ncam)]

    pending_out = [None, None]
    cur = fire(0)
    for g in range(k_chunks):
        buf = g % 2
        for cp in cur:
            cp.wait()
        nxt = None
        if g + 1 < k_chunks:
            nbuf = (g + 1) % 2
            if pending_out[nbuf] is not None:
                pending_out[nbuf].wait()
                pending_out[nbuf] = None
            nxt = fire(g + 1)

        def accum(j, carry, _buf=buf):
            for s2 in range(2):
                sl = pl.ds(s2 * 32, 32)
                acc = rows[_buf][0][j, sl]
                for c in range(1, ncam):
                    acc = acc + rows[_buf][c][j, sl]
                rows[_buf][0][j, sl] = acc
            return carry

        lax.fori_loop(0, V_CHUNK, accum, 0, unroll=2)
        pending_out[buf] = pltpu.async_copy(
            rows[buf][0], out_hbm.at[pl.ds(base + g * V_CHUNK, V_CHUNK)],
            osems[buf])
        cur = nxt
    for po in pending_out:
        if po is not None:
            po.wait()


def kernel(voxel_features, voxel_coords, img_features, rots, trans, intrins,
           post_rots, post_trans, bda, lidar2cam, imgs, W_compress, W_fuse):
    n = voxel_features.shape[0]
    ncam = img_features.shape[1]
    fh, fw = img_features.shape[3], img_features.shape[4]
    p = fh * fw
    out_ch = W_fuse.shape[0]
    in_ch = voxel_features.shape[1]
    cmp_ch = W_compress.shape[0]
    zrow = ncam * p

    try:
        info = plsc.get_sparse_core_info()
        nc, ns = info.num_cores, info.num_subcores
    except Exception:
        nc, ns = 2, 16
    nw = nc * ns
    chunk_rows = nw * V_CHUNK
    k_chunks = -(-n // chunk_rows)
    n_pad = k_chunks * chunk_rows
    nb = n_pad // LANES

    b = 0
    pts = voxel_coords[:, jnp.array([3, 2, 1])].astype(jnp.float32)
    pts = pts * jnp.asarray(VOXEL_SIZE, jnp.float32)[None, :] \
        + jnp.asarray(PC_RANGE, jnp.float32)[None, :]
    bda_b = bda[b]
    pc = pts - bda_b[:3, 3][None, :]
    pc = pc @ jnp.linalg.inv(bda_b[:3, :3]).T
    pis = []
    for c in range(ncam):
        l2c = lidar2cam[b, c]
        cam2img = jnp.eye(4, dtype=jnp.float32).at[:3, :3].set(intrins[b, c])
        lidar2img = cam2img @ l2c.T
        pi = pc @ lidar2img[:3, :3].T + lidar2img[:3, 3][None, :]
        pi = jnp.concatenate([pi[:, :2] / pi[:, 2:3], pi[:, 2:3]], axis=1)
        pi = pi @ post_rots[b, c].T + post_trans[b, c][None, :]
        pis.append(pi)
    pi_t = jnp.transpose(jnp.stack(pis), (0, 2, 1)).reshape(3 * ncam, n)
    pi_t = jnp.pad(pi_t, ((0, 0), (0, n_pad - n)))
    pi3 = pi_t.reshape(3 * ncam, nb, LANES)

    br = 32
    grid_a = nb // br
    assert grid_a * br == nb
    idx3 = pl.pallas_call(
        functools.partial(_idx_body, ncam, fh, fw, zrow),
        grid=(grid_a,),
        in_specs=[
            pl.BlockSpec((3 * ncam, br, LANES), lambda i: (0, i, 0)),
        ],
        out_specs=pl.BlockSpec((ncam, br, LANES), lambda i: (0, i, 0)),
        out_shape=jax.ShapeDtypeStruct((ncam, nb, LANES), jnp.int32),
    )(pi3)
    idx = idx3.reshape(ncam, n_pad)

    img_flat = img_features[b].reshape(ncam, img_features.shape[2], p)
    tbl = pl.pallas_call(
        _table_body,
        grid=(ncam,),
        in_specs=[
            pl.BlockSpec((1, img_flat.shape[1], p), lambda i: (i, 0, 0)),
            pl.BlockSpec((cmp_ch, img_flat.shape[1]), lambda i: (0, 0)),
        ],
        out_specs=pl.BlockSpec((1, p, cmp_ch), lambda i: (i, 0, 0)),
        out_shape=jax.ShapeDtypeStruct((ncam, p, cmp_ch), jnp.bfloat16),
    )(img_flat, W_compress)
    table = jnp.concatenate(
        [tbl.reshape(ncam * p, cmp_ch),
         jnp.zeros((NZROWS, cmp_ch), jnp.bfloat16)], axis=0)

    mesh = plsc.VectorSubcoreMesh(core_axis_name="c", subcore_axis_name="s",
                                  num_cores=nc, num_subcores=ns)
    img_feat = pl.kernel(
        functools.partial(_sc_gather_body, ncam, nc, ns, k_chunks),
        out_type=jax.ShapeDtypeStruct((n_pad, cmp_ch), jnp.bfloat16),
        mesh=mesh,
        scratch_types=[
            pltpu.VMEM((ncam, k_chunks * V_CHUNK), jnp.int32),
            [[pltpu.VMEM((V_CHUNK, cmp_ch), jnp.bfloat16) for _ in range(ncam)]
             for _ in range(2)],
            [[pltpu.SemaphoreType.DMA for _ in range(ncam)] for _ in range(2)],
            [pltpu.SemaphoreType.DMA for _ in range(2)],
            pltpu.VMEM_SHARED((ncam * p + NZROWS, cmp_ch), jnp.bfloat16),
        ],
        compiler_params=pltpu.CompilerParams(use_tc_tiling_on_sc=False),
    )(idx, table)

    w1t = W_fuse[:, :in_ch].T
    w2t = W_fuse[:, in_ch:].T.astype(jnp.bfloat16)
    bn = 512
    grid_c = -(-n // bn)
    fused = pl.pallas_call(
        _fuse_body,
        grid=(grid_c,),
        in_specs=[
            pl.BlockSpec((bn, in_ch), lambda i: (i, 0)),
            pl.BlockSpec((bn, cmp_ch), lambda i: (i, 0)),
            pl.BlockSpec((in_ch, out_ch), lambda i: (0, 0)),
            pl.BlockSpec((cmp_ch, out_ch), lambda i: (0, 0)),
        ],
        out_specs=pl.BlockSpec((bn, out_ch), lambda i: (i, 0)),
        out_shape=jax.ShapeDtypeStruct((n, out_ch), jnp.float32),
    )(voxel_features, img_feat, w1t, w2t)

    return (fused, voxel_coords)
